# baseline (speedup 1.0000x reference)
import functools

import jax
import jax.numpy as jnp
from jax import lax
from jax.experimental import pallas as pl
from jax.experimental.pallas import tpu as pltpu
from jax.experimental.pallas import tpu_sc as plsc

N_NODES = 10000
N_GRAPHS = 64
HIDDEN = 512

NCORES = 2
NW = 32
ROWS_PER_W = 312
BLK = 104
TAIL = 16
TAIL_BASE = NW * ROWS_PER_W

_mesh = plsc.VectorSubcoreMesh(core_axis_name="c", subcore_axis_name="s")


@functools.partial(
    pl.kernel,
    out_type=jax.ShapeDtypeStruct((NCORES, N_GRAPHS, 128), jnp.float32),
    mesh=_mesh,
    scratch_types=[
        pltpu.VMEM((BLK, 128), jnp.float32),
        pltpu.VMEM((TAIL, 128), jnp.float32),
        pltpu.VMEM((3, BLK), jnp.int32),
        pltpu.VMEM((TAIL,), jnp.int32),
        pltpu.VMEM((N_GRAPHS, 128), jnp.float32),
        pltpu.VMEM_SHARED((N_GRAPHS, 128), jnp.float32),
    ],
)
def _sc_segment_sums(x_hbm, ids_hbm, sums_hbm,
                     xbuf, xtail, idsb, idstail, zsums, shsums):
    cid = lax.axis_index("c")
    sid = lax.axis_index("s")
    w = sid * NCORES + cid
    base = w * ROWS_PER_W
    zero16 = jnp.zeros((16,), jnp.float32)

    @pl.when(sid == 0)
    def _():
        @pl.loop(0, N_GRAPHS)
        def _(r):
            @pl.loop(0, 128, step=16)
            def _(c2):
                zsums.at[r, pl.ds(c2, 16)][...] = zero16

        pltpu.sync_copy(zsums, shsums)

    plsc.subcore_barrier()

    for j in range(3):
        start = base + j * BLK
        pltpu.sync_copy(ids_hbm.at[pl.ds(start, BLK)], idsb.at[j])
        pltpu.sync_copy(x_hbm.at[pl.ds(start, BLK)], xbuf)
        pltpu.sync_copy(xbuf, shsums.at[idsb.at[j]], add=True)

    @pl.when(w == 0)
    def _():
        pltpu.sync_copy(ids_hbm.at[pl.ds(TAIL_BASE, TAIL)], idstail)
        pltpu.sync_copy(x_hbm.at[pl.ds(TAIL_BASE, TAIL)], xtail)
        pltpu.sync_copy(xtail, shsums.at[idstail], add=True)

    plsc.subcore_barrier()

    @pl.when(sid == 0)
    def _():
        pltpu.sync_copy(shsums, sums_hbm.at[cid])


def _tc_body(sp_ref, b_ref, u_ref, w1u_ref, w1x_ref, b1_ref, w2_ref, b2_ref,
             w3_ref, b3_ref, out_ref):
    sums = sp_ref[0] + sp_ref[1]
    seg_iota = lax.broadcasted_iota(jnp.int32, (N_NODES, N_GRAPHS), 1)
    onehot = (b_ref[...] == seg_iota).astype(jnp.float32)
    cnt = jnp.sum(onehot, axis=0)[:, None]
    mean = sums / jnp.maximum(cnt, 1.0)
    h = (u_ref[...] @ w1u_ref[...]
         + lax.dot_general(mean, w1x_ref[...], (((1,), (0,)), ((), ())),
                           preferred_element_type=jnp.float32,
                           precision=lax.Precision.HIGHEST)
         + b1_ref[...])
    h = jnp.maximum(h, 0.0)
    h = lax.dot_general(h, w2_ref[...], (((1,), (0,)), ((), ())),
                        preferred_element_type=jnp.float32,
                        precision=lax.Precision.HIGHEST) + b2_ref[...]
    h = jnp.maximum(h, 0.0)
    out_ref[...] = lax.dot_general(h, w3_ref[...], (((1,), (0,)), ((), ())),
                                   preferred_element_type=jnp.float32,
                                   precision=lax.Precision.HIGHEST) + b3_ref[...]


def kernel(x, edge_index, edge_attr, u, batch, W1, b1, W2, b2, W3, b3):
    del edge_index, edge_attr
    batch32 = batch.astype(jnp.int32)
    sums_p = _sc_segment_sums(x, batch32)
    u2 = u.reshape(N_GRAPHS, 6)
    W1u = W1[:6]
    W1x = W1[6:]
    return pl.pallas_call(
        _tc_body,
        out_shape=jax.ShapeDtypeStruct((N_GRAPHS, 128), jnp.float32),
    )(sums_p, batch32.reshape(N_NODES, 1), u2, W1u, W1x,
      b1.reshape(1, HIDDEN), W2, b2.reshape(1, HIDDEN), W3,
      b3.reshape(1, 128))


# speedup vs baseline: 1.0857x; 1.0857x over previous
import functools

import jax
import jax.numpy as jnp
from jax import lax
from jax.experimental import pallas as pl
from jax.experimental.pallas import tpu as pltpu
from jax.experimental.pallas import tpu_sc as plsc

N_NODES = 10000
N_GRAPHS = 64
HIDDEN = 512

NCORES = 2
NSUB = 16
NW = 32
ROWS_PER_W = 312
BLK = 104
NBLK = 3
TAIL = 16
TAIL_BASE = NW * ROWS_PER_W
ZROWS = N_GRAPHS // NSUB

_mesh = plsc.VectorSubcoreMesh(core_axis_name="c", subcore_axis_name="s")


@functools.partial(
    pl.kernel,
    out_type=jax.ShapeDtypeStruct((NCORES, N_GRAPHS, 128), jnp.float32),
    mesh=_mesh,
    scratch_types=[
        pltpu.VMEM((2, BLK, 128), jnp.float32),
        pltpu.VMEM((TAIL, 128), jnp.float32),
        pltpu.VMEM((NBLK, BLK), jnp.int32),
        pltpu.VMEM((TAIL,), jnp.int32),
        pltpu.VMEM((ZROWS, 128), jnp.float32),
        pltpu.VMEM_SHARED((N_GRAPHS, 128), jnp.float32),
        pltpu.SemaphoreType.DMA,
        pltpu.SemaphoreType.DMA,
    ],
)
def _sc_segment_sums(x_hbm, ids_hbm, sums_hbm,
                     xbuf, xtail, idsb, idstail, zsums, shsums, sem0, sem1):
    cid = lax.axis_index("c")
    sid = lax.axis_index("s")
    w = sid * NCORES + cid
    base = w * ROWS_PER_W
    zero16 = jnp.zeros((16,), jnp.float32)
    sems = (sem0, sem1)

    @pl.loop(0, ZROWS)
    def _(r):
        @pl.loop(0, 128, step=16)
        def _(c2):
            zsums.at[r, pl.ds(c2, 16)][...] = zero16

    pltpu.sync_copy(zsums, shsums.at[pl.ds(sid * ZROWS, ZROWS)])

    for j in range(NBLK):
        pltpu.sync_copy(ids_hbm.at[pl.ds(base + j * BLK, BLK)], idsb.at[j])

    plsc.subcore_barrier()

    loads = [
        pltpu.make_async_copy(x_hbm.at[pl.ds(base + j * BLK, BLK)],
                              xbuf.at[j % 2], sems[j % 2])
        for j in range(NBLK)
    ]
    loads[0].start()
    for j in range(NBLK):
        loads[j].wait()
        if j + 1 < NBLK:
            loads[j + 1].start()
        pltpu.sync_copy(xbuf.at[j % 2], shsums.at[idsb.at[j]], add=True)

    @pl.when(w == 0)
    def _():
        pltpu.sync_copy(ids_hbm.at[pl.ds(TAIL_BASE, TAIL)], idstail)
        pltpu.sync_copy(x_hbm.at[pl.ds(TAIL_BASE, TAIL)], xtail)
        pltpu.sync_copy(xtail, shsums.at[idstail], add=True)

    plsc.subcore_barrier()

    @pl.when(sid == 0)
    def _():
        pltpu.sync_copy(shsums, sums_hbm.at[cid])


def _tc_counts_body(b_ref, cnt_ref):
    seg_iota = lax.broadcasted_iota(jnp.int32, (N_NODES, N_GRAPHS), 1)
    onehot = (b_ref[...] == seg_iota).astype(jnp.float32)
    cnt_ref[...] = jnp.sum(onehot, axis=0)[:, None]


def _tc_mlp_body(sp_ref, cnt_ref, u_ref, w1u_ref, w1x_ref, b1_ref, w2_ref,
                 b2_ref, w3_ref, b3_ref, out_ref):
    sums = sp_ref[0] + sp_ref[1]
    mean = sums / jnp.maximum(cnt_ref[...], 1.0)
    h = (u_ref[...] @ w1u_ref[...]
         + lax.dot_general(mean, w1x_ref[...], (((1,), (0,)), ((), ())),
                           preferred_element_type=jnp.float32,
                           precision=lax.Precision.HIGHEST)
         + b1_ref[...])
    h = jnp.maximum(h, 0.0)
    h = lax.dot_general(h, w2_ref[...], (((1,), (0,)), ((), ())),
                        preferred_element_type=jnp.float32,
                        precision=lax.Precision.HIGHEST) + b2_ref[...]
    h = jnp.maximum(h, 0.0)
    out_ref[...] = lax.dot_general(h, w3_ref[...], (((1,), (0,)), ((), ())),
                                   preferred_element_type=jnp.float32,
                                   precision=lax.Precision.HIGHEST) + b3_ref[...]


def kernel(x, edge_index, edge_attr, u, batch, W1, b1, W2, b2, W3, b3):
    del edge_index, edge_attr
    batch32 = batch.astype(jnp.int32)
    sums_p = _sc_segment_sums(x, batch32)
    cnt = pl.pallas_call(
        _tc_counts_body,
        out_shape=jax.ShapeDtypeStruct((N_GRAPHS, 1), jnp.float32),
    )(batch32.reshape(N_NODES, 1))
    u2 = u.reshape(N_GRAPHS, 6)
    W1u = W1[:6]
    W1x = W1[6:]
    return pl.pallas_call(
        _tc_mlp_body,
        out_shape=jax.ShapeDtypeStruct((N_GRAPHS, 128), jnp.float32),
    )(sums_p, cnt, u2, W1u, W1x, b1.reshape(1, HIDDEN), W2,
      b2.reshape(1, HIDDEN), W3, b3.reshape(1, 128))


# speedup vs baseline: 1.2664x; 1.1665x over previous
import functools

import jax
import jax.numpy as jnp
from jax import lax
from jax.experimental import pallas as pl
from jax.experimental.pallas import tpu as pltpu
from jax.experimental.pallas import tpu_sc as plsc

N_NODES = 10000
N_GRAPHS = 64
HIDDEN = 512

NCORES = 2
NSUB = 16
NW = 32
ROWS_PER_W = 312
BLK = 104
NBLK = 3
TAIL = 16
TAIL_BASE = NW * ROWS_PER_W
ZROWS = N_GRAPHS // NSUB

_mesh = plsc.VectorSubcoreMesh(core_axis_name="c", subcore_axis_name="s")


@functools.partial(
    pl.kernel,
    out_type=jax.ShapeDtypeStruct((NCORES, N_GRAPHS, 128), jnp.float32),
    mesh=_mesh,
    scratch_types=[
        pltpu.VMEM((NBLK, BLK, 128), jnp.float32),
        pltpu.VMEM((TAIL, 128), jnp.float32),
        pltpu.VMEM((NBLK, BLK), jnp.int32),
        pltpu.VMEM((TAIL,), jnp.int32),
        pltpu.VMEM((ZROWS, 128), jnp.float32),
        pltpu.VMEM_SHARED((N_GRAPHS, 128), jnp.float32),
        pltpu.SemaphoreType.DMA,
        pltpu.SemaphoreType.DMA,
        pltpu.SemaphoreType.DMA,
        pltpu.SemaphoreType.DMA,
        pltpu.SemaphoreType.DMA,
    ],
)
def _sc_segment_sums(x_hbm, ids_hbm, sums_hbm,
                     xbuf, xtail, idsb, idstail, zsums, shsums,
                     semi, semx0, semx1, semx2, sems):
    cid = lax.axis_index("c")
    sid = lax.axis_index("s")
    w = sid * NCORES + cid
    base = w * ROWS_PER_W
    zero16 = jnp.zeros((16,), jnp.float32)
    semx = (semx0, semx1, semx2)

    loads = []
    idloads = []
    for j in range(NBLK):
        loads.append(pltpu.async_copy(
            x_hbm.at[pl.ds(base + j * BLK, BLK)], xbuf.at[j], semx[j]))
        idloads.append(pltpu.async_copy(
            ids_hbm.at[pl.ds(base + j * BLK, BLK)], idsb.at[j], semi))

    @pl.loop(0, ZROWS)
    def _(r):
        @pl.loop(0, 128, step=16)
        def _(c2):
            zsums.at[r, pl.ds(c2, 16)][...] = zero16

    pltpu.sync_copy(zsums, shsums.at[pl.ds(sid * ZROWS, ZROWS)])
    for j in range(NBLK):
        idloads[j].wait()
    plsc.subcore_barrier()

    scatters = []
    for j in range(NBLK):
        loads[j].wait()
        scatters.append(pltpu.async_copy(
            xbuf.at[j], shsums.at[idsb.at[j]], sems, add=True))

    @pl.when(w == 0)
    def _():
        pltpu.sync_copy(ids_hbm.at[pl.ds(TAIL_BASE, TAIL)], idstail)
        pltpu.sync_copy(x_hbm.at[pl.ds(TAIL_BASE, TAIL)], xtail)
        pltpu.sync_copy(xtail, shsums.at[idstail], add=True)

    for j in range(NBLK):
        scatters[j].wait()
    plsc.subcore_barrier()

    @pl.when(sid == 0)
    def _():
        pltpu.sync_copy(shsums, sums_hbm.at[cid])


def _tc_body(sp_ref, b_ref, u_ref, w1u_ref, w1x_ref, b1_ref, w2_ref,
             b2_ref, w3_ref, b3_ref, out_ref):
    seg_iota = lax.broadcasted_iota(jnp.int32, (N_GRAPHS, N_NODES), 0)
    onehot = (b_ref[...][None, :] == seg_iota).astype(jnp.float32)
    cnt = jnp.sum(onehot, axis=1)[:, None]
    sums = sp_ref[0] + sp_ref[1]
    mean = sums / jnp.maximum(cnt, 1.0)
    h = (u_ref[...] @ w1u_ref[...]
         + lax.dot_general(mean, w1x_ref[...], (((1,), (0,)), ((), ())),
                           preferred_element_type=jnp.float32)
         + b1_ref[...])
    h = jnp.maximum(h, 0.0)
    h = lax.dot_general(h, w2_ref[...], (((1,), (0,)), ((), ())),
                        preferred_element_type=jnp.float32) + b2_ref[...]
    h = jnp.maximum(h, 0.0)
    out_ref[...] = lax.dot_general(h, w3_ref[...], (((1,), (0,)), ((), ())),
                                   preferred_element_type=jnp.float32
                                   ) + b3_ref[...]


def kernel(x, edge_index, edge_attr, u, batch, W1, b1, W2, b2, W3, b3):
    del edge_index, edge_attr
    batch32 = batch.astype(jnp.int32)
    sums_p = _sc_segment_sums(x, batch32)
    u2 = u.reshape(N_GRAPHS, 6)
    W1u = W1[:6]
    W1x = W1[6:]
    return pl.pallas_call(
        _tc_body,
        out_shape=jax.ShapeDtypeStruct((N_GRAPHS, 128), jnp.float32),
    )(sums_p, batch32, u2, W1u, W1x, b1.reshape(1, HIDDEN), W2,
      b2.reshape(1, HIDDEN), W3, b3.reshape(1, 128))


# speedup vs baseline: 1.3724x; 1.0837x over previous
import functools

import jax
import jax.numpy as jnp
from jax import lax
from jax.experimental import pallas as pl
from jax.experimental.pallas import tpu as pltpu
from jax.experimental.pallas import tpu_sc as plsc

N_NODES = 10000
N_GRAPHS = 64
HIDDEN = 512

NCORES = 2
NSUB = 16
NW = 32
ROWS_PER_W = 208
BLK = 104
NBLK = 2
SC_ROWS = NW * ROWS_PER_W
TC_ROWS = 3328
TAIL = 16
TAIL_BASE = SC_ROWS + TC_ROWS
ZROWS = N_GRAPHS // NSUB

_mesh = plsc.VectorSubcoreMesh(core_axis_name="c", subcore_axis_name="s")


@functools.partial(
    pl.kernel,
    out_type=jax.ShapeDtypeStruct((NCORES, N_GRAPHS, 128), jnp.float32),
    mesh=_mesh,
    scratch_types=[
        pltpu.VMEM((NBLK, BLK, 128), jnp.float32),
        pltpu.VMEM((TAIL, 128), jnp.float32),
        pltpu.VMEM((NBLK, BLK), jnp.int32),
        pltpu.VMEM((TAIL,), jnp.int32),
        pltpu.VMEM((ZROWS, 128), jnp.float32),
        pltpu.VMEM_SHARED((N_GRAPHS, 128), jnp.float32),
        pltpu.SemaphoreType.DMA,
        pltpu.SemaphoreType.DMA,
        pltpu.SemaphoreType.DMA,
        pltpu.SemaphoreType.DMA,
    ],
)
def _sc_segment_sums(x_hbm, ids_hbm, sums_hbm,
                     xbuf, xtail, idsb, idstail, zsums, shsums,
                     semi, semx0, semx1, sems):
    cid = lax.axis_index("c")
    sid = lax.axis_index("s")
    w = sid * NCORES + cid
    base = w * ROWS_PER_W
    zero16 = jnp.zeros((16,), jnp.float32)
    semx = (semx0, semx1)

    loads = []
    idloads = []
    for j in range(NBLK):
        loads.append(pltpu.async_copy(
            x_hbm.at[pl.ds(base + j * BLK, BLK)], xbuf.at[j], semx[j]))
        idloads.append(pltpu.async_copy(
            ids_hbm.at[pl.ds(base + j * BLK, BLK)], idsb.at[j], semi))

    @pl.loop(0, ZROWS)
    def _(r):
        @pl.loop(0, 128, step=16)
        def _(c2):
            zsums.at[r, pl.ds(c2, 16)][...] = zero16

    pltpu.sync_copy(zsums, shsums.at[pl.ds(sid * ZROWS, ZROWS)])
    for j in range(NBLK):
        idloads[j].wait()
    plsc.subcore_barrier()

    scatters = []
    for j in range(NBLK):
        loads[j].wait()
        scatters.append(pltpu.async_copy(
            xbuf.at[j], shsums.at[idsb.at[j]], sems, add=True))

    @pl.when(w == 0)
    def _():
        pltpu.sync_copy(ids_hbm.at[pl.ds(TAIL_BASE, TAIL)], idstail)
        pltpu.sync_copy(x_hbm.at[pl.ds(TAIL_BASE, TAIL)], xtail)
        pltpu.sync_copy(xtail, shsums.at[idstail], add=True)

    for j in range(NBLK):
        scatters[j].wait()
    plsc.subcore_barrier()

    @pl.when(sid == 0)
    def _():
        pltpu.sync_copy(shsums, sums_hbm.at[cid])


def _tc_partial_body(x_ref, b_ref, out_ref):
    seg_iota = lax.broadcasted_iota(jnp.int32, (N_GRAPHS, TC_ROWS), 0)
    onehot_t = (b_ref[...][None, :] == seg_iota).astype(jnp.float32)
    out_ref[...] = lax.dot_general(
        onehot_t, x_ref[...], (((1,), (0,)), ((), ())),
        preferred_element_type=jnp.float32)


def _tc_mlp_body(sp_ref, tp_ref, b_ref, u_ref, w1u_ref, w1x_ref, b1_ref,
                 w2_ref, b2_ref, w3_ref, b3_ref, out_ref):
    seg_iota = lax.broadcasted_iota(jnp.int32, (N_GRAPHS, N_NODES), 0)
    onehot = (b_ref[...][None, :] == seg_iota).astype(jnp.float32)
    cnt = jnp.sum(onehot, axis=1)[:, None]
    sums = sp_ref[0] + sp_ref[1] + tp_ref[...]
    mean = sums / jnp.maximum(cnt, 1.0)
    h = (u_ref[...] @ w1u_ref[...]
         + lax.dot_general(mean, w1x_ref[...], (((1,), (0,)), ((), ())),
                           preferred_element_type=jnp.float32)
         + b1_ref[...])
    h = jnp.maximum(h, 0.0)
    h = lax.dot_general(h, w2_ref[...], (((1,), (0,)), ((), ())),
                        preferred_element_type=jnp.float32) + b2_ref[...]
    h = jnp.maximum(h, 0.0)
    out_ref[...] = lax.dot_general(h, w3_ref[...], (((1,), (0,)), ((), ())),
                                   preferred_element_type=jnp.float32
                                   ) + b3_ref[...]


def kernel(x, edge_index, edge_attr, u, batch, W1, b1, W2, b2, W3, b3):
    del edge_index, edge_attr
    batch32 = batch.astype(jnp.int32)
    sums_p = _sc_segment_sums(x, batch32)
    ids_tc = lax.slice(batch32, (SC_ROWS,), (SC_ROWS + TC_ROWS,))
    tc_part = pl.pallas_call(
        _tc_partial_body,
        grid=(1,),
        in_specs=[
            pl.BlockSpec((TC_ROWS, 128), lambda i: (SC_ROWS // TC_ROWS, 0)),
            pl.BlockSpec((TC_ROWS,), lambda i: (0,)),
        ],
        out_specs=pl.BlockSpec((N_GRAPHS, 128), lambda i: (0, 0)),
        out_shape=jax.ShapeDtypeStruct((N_GRAPHS, 128), jnp.float32),
    )(x, ids_tc)
    u2 = u.reshape(N_GRAPHS, 6)
    W1u = W1[:6]
    W1x = W1[6:]
    return pl.pallas_call(
        _tc_mlp_body,
        out_shape=jax.ShapeDtypeStruct((N_GRAPHS, 128), jnp.float32),
    )(sums_p, tc_part, batch32, u2, W1u, W1x, b1.reshape(1, HIDDEN), W2,
      b2.reshape(1, HIDDEN), W3, b3.reshape(1, 128))
